# baseline (speedup 1.0000x reference)
import functools

import jax
import jax.numpy as jnp
from jax import lax
from jax.experimental import pallas as pl
from jax.experimental.pallas import tpu as pltpu
from jax.experimental.pallas import tpu_sc as plsc

CHUNK = 128
NC = 2
NS = 16
NW = NC * NS


def _make_gather(n_rows: int):
    assert n_rows % (NW * 2 * CHUNK) == 0
    chunks_per_w = n_rows // (NW * CHUNK)
    n_pairs = chunks_per_w // 2

    mesh = plsc.VectorSubcoreMesh(core_axis_name="c", subcore_axis_name="s")

    @functools.partial(
        pl.kernel,
        mesh=mesh,
        compiler_params=pltpu.CompilerParams(use_tc_tiling_on_sc=False),
        out_type=jax.ShapeDtypeStruct((n_rows, 32), jnp.float32),
        scratch_types=[
            pltpu.VMEM((n_rows // (NW * CHUNK), CHUNK), jnp.int32),
            pltpu.VMEM((CHUNK, 128), jnp.float32),
            pltpu.VMEM((CHUNK, 128), jnp.float32),
            pltpu.SemaphoreType.DMA,
            pltpu.SemaphoreType.DMA,
            pltpu.SemaphoreType.DMA,
            pltpu.SemaphoreType.DMA,
        ],
    )
    def gather(idx_hbm, table_hbm, out_hbm,
               idx_v, rows_a, rows_b, gsem_a, gsem_b, wsem_a, wsem_b):
        wid = lax.axis_index("s") * NC + lax.axis_index("c")
        chunk_base = wid * chunks_per_w
        pltpu.sync_copy(idx_hbm.at[pl.ds(chunk_base, chunks_per_w)], idx_v)

        def fire(c, buf, sem):
            return pltpu.async_copy(table_hbm.at[idx_v.at[c]], buf, sem)

        def write(c, buf, sem):
            return pltpu.async_copy(
                buf.at[:, pl.ds(0, 32)],
                out_hbm.at[pl.ds((chunk_base + c) * CHUNK, CHUNK)],
                sem,
            )

        def pair_body(k, carry):
            c0 = 2 * k
            ha = fire(c0, rows_a, gsem_a)
            hb = fire(c0 + 1, rows_b, gsem_b)
            ha.wait()
            wa = write(c0, rows_a, wsem_a)
            hb.wait()
            wb = write(c0 + 1, rows_b, wsem_b)
            wa.wait()
            wb.wait()
            return carry

        lax.fori_loop(0, n_pairs, pair_body, 0)

    return gather


def kernel(inputs, lookup_table):
    batch, hist = inputs.shape
    n_embed, d = lookup_table.shape
    n_rows = batch * hist
    idx = inputs if inputs.dtype == jnp.int32 else inputs.astype(jnp.int32)
    idx2d = jnp.reshape(idx, (n_rows // CHUNK, CHUNK))
    padded = jnp.pad(lookup_table, ((0, 0), (0, 128 - d)))
    flat = _make_gather(n_rows)(idx2d, padded)
    return jnp.reshape(flat, (batch, hist, d))


# speedup vs baseline: 1.8501x; 1.8501x over previous
import functools

import jax
import jax.numpy as jnp
from jax import lax
from jax.experimental import pallas as pl
from jax.experimental.pallas import tpu as pltpu
from jax.experimental.pallas import tpu_sc as plsc

GROUP = 16
NC = 2
NS = 16
NW = NC * NS


def _make_lookup(n_embed: int, d: int, batch: int, hist: int):
    assert batch % (NW * 2 * GROUP) == 0
    rows_per_w = batch // NW
    n_pairs = rows_per_w // (2 * GROUP)

    mesh = plsc.VectorSubcoreMesh(core_axis_name="c", subcore_axis_name="s")

    @functools.partial(
        pl.kernel,
        mesh=mesh,
        compiler_params=pltpu.CompilerParams(use_tc_tiling_on_sc=False),
        out_type=jax.ShapeDtypeStruct((batch, hist, d), jnp.float32),
        scratch_types=[
            pltpu.VMEM((rows_per_w, hist), jnp.int32),
            pltpu.VMEM((GROUP, hist, d), jnp.float32),
            pltpu.VMEM((GROUP, hist, d), jnp.float32),
            pltpu.SemaphoreType.DMA,
            pltpu.SemaphoreType.DMA,
            pltpu.SemaphoreType.DMA,
            pltpu.SemaphoreType.DMA,
        ],
    )
    def lookup(idx_hbm, table_hbm, out_hbm,
               idx_v, rows_a, rows_b, gsem_a, gsem_b, wsem_a, wsem_b):
        wid = lax.axis_index("s") * NC + lax.axis_index("c")
        row_base = wid * rows_per_w
        pltpu.sync_copy(idx_hbm.at[pl.ds(row_base, rows_per_w)], idx_v)

        def fire(g, buf, sem):
            return [
                pltpu.async_copy(
                    table_hbm.at[idx_v.at[g * GROUP + b]],
                    buf.at[b],
                    sem,
                )
                for b in range(GROUP)
            ]

        def write(g, buf, sem):
            row0 = row_base + g * GROUP
            return pltpu.async_copy(buf, out_hbm.at[pl.ds(row0, GROUP)], sem)

        def pair_body(k, carry):
            g0 = 2 * k
            ha = fire(g0, rows_a, gsem_a)
            hb = fire(g0 + 1, rows_b, gsem_b)
            for h in ha:
                h.wait()
            wa = write(g0, rows_a, wsem_a)
            for h in hb:
                h.wait()
            wb = write(g0 + 1, rows_b, wsem_b)
            wa.wait()
            wb.wait()
            return carry

        lax.fori_loop(0, n_pairs, pair_body, 0)

    return lookup


def kernel(inputs, lookup_table):
    batch, hist = inputs.shape
    n_embed, d = lookup_table.shape
    idx = inputs if inputs.dtype == jnp.int32 else inputs.astype(jnp.int32)
    return _make_lookup(n_embed, d, batch, hist)(idx, lookup_table)


# speedup vs baseline: 2.5958x; 1.4031x over previous
import functools

import jax
import jax.numpy as jnp
from jax import lax
from jax.experimental import pallas as pl
from jax.experimental.pallas import tpu as pltpu
from jax.experimental.pallas import tpu_sc as plsc

GROUP = 16
NC = 2
NS = 16
NW = NC * NS


def _make_lookup(n_embed: int, d: int, batch: int, hist: int):
    assert batch % (NW * 2 * GROUP) == 0
    rows_per_w = batch // NW
    n_pairs = rows_per_w // (2 * GROUP)

    mesh = plsc.VectorSubcoreMesh(core_axis_name="c", subcore_axis_name="s")

    @functools.partial(
        pl.kernel,
        mesh=mesh,
        compiler_params=pltpu.CompilerParams(use_tc_tiling_on_sc=False),
        out_type=jax.ShapeDtypeStruct((batch, (hist + 7) // 8 * 8, 128), jnp.float32),
        scratch_types=[
            pltpu.VMEM((rows_per_w, hist), jnp.int32),
            pltpu.VMEM((GROUP, hist, d), jnp.float32),
            pltpu.VMEM((GROUP, hist, d), jnp.float32),
            pltpu.SemaphoreType.DMA,
            pltpu.SemaphoreType.DMA,
            pltpu.SemaphoreType.DMA,
            pltpu.SemaphoreType.DMA,
        ],
    )
    def lookup(idx_hbm, table_hbm, out_hbm,
               idx_v, rows_a, rows_b, gsem_a, gsem_b, wsem_a, wsem_b):
        wid = lax.axis_index("s") * NC + lax.axis_index("c")
        row_base = wid * rows_per_w
        pltpu.sync_copy(idx_hbm.at[pl.ds(row_base, rows_per_w)], idx_v)

        def fire(g, buf, sem):
            return [
                pltpu.async_copy(
                    table_hbm.at[idx_v.at[g * GROUP + b]],
                    buf.at[b],
                    sem,
                )
                for b in range(GROUP)
            ]

        def write(g, buf, sem):
            row0 = row_base + g * GROUP
            return pltpu.async_copy(
                buf,
                out_hbm.at[pl.ds(row0, GROUP), pl.ds(0, hist), pl.ds(0, d)],
                sem,
            )

        def pair_body(k, carry):
            g0 = 2 * k
            ha = fire(g0, rows_a, gsem_a)
            hb = fire(g0 + 1, rows_b, gsem_b)
            for h in ha:
                h.wait()
            wa = write(g0, rows_a, wsem_a)
            for h in hb:
                h.wait()
            wb = write(g0 + 1, rows_b, wsem_b)
            wa.wait()
            wb.wait()
            return carry

        lax.fori_loop(0, n_pairs, pair_body, 0)

    return lookup


def kernel(inputs, lookup_table):
    batch, hist = inputs.shape
    n_embed, d = lookup_table.shape
    idx = inputs if inputs.dtype == jnp.int32 else inputs.astype(jnp.int32)
    raw = _make_lookup(n_embed, d, batch, hist)(idx, lookup_table)
    return raw[:, :hist, :d]


# speedup vs baseline: 2.6061x; 1.0040x over previous
import functools

import jax
import jax.numpy as jnp
from jax import lax
from jax.experimental import pallas as pl
from jax.experimental.pallas import tpu as pltpu
from jax.experimental.pallas import tpu_sc as plsc

GROUP = 32
NC = 2
NS = 16
NW = NC * NS


def _make_lookup(n_embed: int, d: int, batch: int, hist: int):
    assert batch % (NW * 2 * GROUP) == 0
    rows_per_w = batch // NW
    n_pairs = rows_per_w // (2 * GROUP)

    mesh = plsc.VectorSubcoreMesh(core_axis_name="c", subcore_axis_name="s")

    @functools.partial(
        pl.kernel,
        mesh=mesh,
        compiler_params=pltpu.CompilerParams(use_tc_tiling_on_sc=False),
        out_type=jax.ShapeDtypeStruct((batch, (hist + 7) // 8 * 8, 128), jnp.float32),
        scratch_types=[
            pltpu.VMEM((rows_per_w, hist), jnp.int32),
            pltpu.VMEM((GROUP, hist, d), jnp.float32),
            pltpu.VMEM((GROUP, hist, d), jnp.float32),
            pltpu.SemaphoreType.DMA,
            pltpu.SemaphoreType.DMA,
            pltpu.SemaphoreType.DMA,
            pltpu.SemaphoreType.DMA,
        ],
    )
    def lookup(idx_hbm, table_hbm, out_hbm,
               idx_v, rows_a, rows_b, gsem_a, gsem_b, wsem_a, wsem_b):
        wid = lax.axis_index("s") * NC + lax.axis_index("c")
        row_base = wid * rows_per_w
        pltpu.sync_copy(idx_hbm.at[pl.ds(row_base, rows_per_w)], idx_v)

        def fire(g, buf, sem):
            return [
                pltpu.async_copy(
                    table_hbm.at[idx_v.at[g * GROUP + b]],
                    buf.at[b],
                    sem,
                )
                for b in range(GROUP)
            ]

        def write(g, buf, sem):
            row0 = row_base + g * GROUP
            return pltpu.async_copy(
                buf,
                out_hbm.at[pl.ds(row0, GROUP), pl.ds(0, hist), pl.ds(0, d)],
                sem,
            )

        def pair_body(k, carry):
            g0 = 2 * k
            ha = fire(g0, rows_a, gsem_a)
            hb = fire(g0 + 1, rows_b, gsem_b)
            for h in ha:
                h.wait()
            wa = write(g0, rows_a, wsem_a)
            for h in hb:
                h.wait()
            wb = write(g0 + 1, rows_b, wsem_b)
            wa.wait()
            wb.wait()
            return carry

        lax.fori_loop(0, n_pairs, pair_body, 0)

    return lookup


def kernel(inputs, lookup_table):
    batch, hist = inputs.shape
    n_embed, d = lookup_table.shape
    idx = inputs if inputs.dtype == jnp.int32 else inputs.astype(jnp.int32)
    raw = _make_lookup(n_embed, d, batch, hist)(idx, lookup_table)
    return raw[:, :hist, :d]
